# baseline (speedup 1.0000x reference)
import functools

import jax
import jax.numpy as jnp
from jax import lax
from jax.experimental import pallas as pl
from jax.experimental.pallas import tpu as pltpu
from jax.experimental.pallas import tpu_sc as plsc

N, D, B = 32768, 1024, 16
NC, NS = 2, 16
NW = NC * NS
CHUNK = N // NW
RB = 32
NBLK = CHUNK // RB
LANES = 16


def _sc_partials(x, run_lo, run_n):
    mesh = plsc.VectorSubcoreMesh(core_axis_name="c", subcore_axis_name="s")

    @functools.partial(
        pl.kernel,
        out_type=jax.ShapeDtypeStruct((NW, B, D), jnp.float32),
        mesh=mesh,
        scratch_types=[
            pltpu.VMEM((2, RB, D), jnp.float32),
            pltpu.VMEM((B, D), jnp.float32),
            pltpu.VMEM((LANES,), jnp.int32),
            pltpu.VMEM((LANES,), jnp.int32),
            pltpu.SMEM((B,), jnp.int32),
            pltpu.SMEM((B,), jnp.int32),
            pltpu.SemaphoreType.DMA,
        ],
        compiler_params=pltpu.CompilerParams(
            use_tc_tiling_on_sc=False, needs_layout_passes=False
        ),
    )
    def k(x_hbm, lo_hbm, n_hbm, out_hbm, buf, acc, lo_v, n_v, lo_s, n_s, sem):
        c = lax.axis_index("c")
        s = lax.axis_index("s")
        w = s * NC + c
        base = w * CHUNK

        pltpu.sync_copy(lo_hbm.at[w], lo_v)
        pltpu.sync_copy(n_hbm.at[w], n_v)

        lanes = lax.iota(jnp.int32, LANES)
        lo_all = lo_v[...]
        n_all = n_v[...]

        def extract(j, _):
            lo_s[j] = jnp.sum(jnp.where(lanes == j, lo_all, 0)) - base
            n_s[j] = jnp.sum(jnp.where(lanes == j, n_all, 0))
            return 0

        lax.fori_loop(0, B, extract, 0)

        zeros = jnp.zeros((LANES,), jnp.float32)

        def zero_b(b, _):
            def zero_cs(cs, __):
                acc[b, pl.ds(cs * LANES, LANES)] = zeros
                return 0

            return lax.fori_loop(0, D // LANES, zero_cs, 0)

        lax.fori_loop(0, B, zero_b, 0)

        def block_copy(blk, slot):
            return pltpu.make_async_copy(
                x_hbm.at[pl.ds(base + blk * RB, RB), :],
                buf.at[slot],
                sem,
            )

        block_copy(0, 0).start()
        UNROLL = 8

        def blk_body(blk, _):
            slot = lax.rem(blk, 2)
            block_copy(blk, slot).wait()

            @pl.when(blk + 1 < NBLK)
            def _():
                block_copy(blk + 1, 1 - slot).start()

            wlo = blk * RB

            def j_body(j, __):
                lo_j = lo_s[j]
                n_j = n_s[j]
                lo_w = jnp.maximum(lo_j, wlo)
                hi_w = jnp.minimum(lo_j + n_j, wlo + RB)
                n_w = hi_w - lo_w
                r0 = lo_w - wlo

                @pl.when(n_w > 0)
                def _():
                    nu = n_w - lax.rem(n_w, UNROLL)

                    def cs_body(cs, __):
                        c0 = cs * (2 * LANES)

                        def body_u(kk, carry):
                            a0, a1 = carry
                            r = r0 + kk * UNROLL
                            for t in range(UNROLL):
                                a0 = a0 + buf[slot, r + t, pl.ds(c0, LANES)]
                                a1 = a1 + buf[
                                    slot, r + t, pl.ds(c0 + LANES, LANES)
                                ]
                            return (a0, a1)

                        a0, a1 = lax.fori_loop(
                            0, nu // UNROLL, body_u, (zeros, zeros)
                        )

                        def body_rem(i, carry):
                            a0, a1 = carry
                            a0 = a0 + buf[slot, r0 + i, pl.ds(c0, LANES)]
                            a1 = a1 + buf[
                                slot, r0 + i, pl.ds(c0 + LANES, LANES)
                            ]
                            return (a0, a1)

                        a0, a1 = lax.fori_loop(nu, n_w, body_rem, (a0, a1))
                        plsc.addupdate(acc.at[j, pl.ds(c0, LANES)], a0)
                        plsc.addupdate(acc.at[j, pl.ds(c0 + LANES, LANES)], a1)
                        return 0

                    lax.fori_loop(0, D // (2 * LANES), cs_body, 0)

                return 0

            return lax.fori_loop(0, B, j_body, 0)

        lax.fori_loop(0, NBLK, blk_body, 0)

        pltpu.sync_copy(acc, out_hbm.at[w])

    return k(x, run_lo, run_n)


def _combine(partials, inv_len):
    def body(p_ref, inv_ref, o_ref):
        o_ref[...] = jnp.sum(p_ref[...], axis=0) * inv_ref[...]

    return pl.pallas_call(
        body,
        out_shape=jax.ShapeDtypeStruct((B, D), jnp.float32),
    )(partials, inv_len)


def kernel(x, batch_lengths):
    ends = jnp.cumsum(batch_lengths, dtype=jnp.int32)
    starts = jnp.concatenate([jnp.zeros((1,), jnp.int32), ends[:-1]])

    wlo = jnp.arange(NW, dtype=jnp.int32)[:, None] * CHUNK
    whi = wlo + CHUNK
    lo = jnp.maximum(starts[None, :], wlo)
    hi = jnp.minimum(ends[None, :], whi)
    n = jnp.maximum(hi - lo, 0)

    partials = _sc_partials(x, lo, n)
    inv_len = (1.0 / batch_lengths.astype(jnp.float32))[:, None]
    return _combine(partials, inv_len)


# speedup vs baseline: 1.9372x; 1.9372x over previous
import functools

import jax
import jax.numpy as jnp
from jax import lax
from jax.experimental import pallas as pl
from jax.experimental.pallas import tpu as pltpu
from jax.experimental.pallas import tpu_sc as plsc

N, D, B = 32768, 1024, 16
NC, NS = 2, 16
NW = NC * NS
CHUNK = N // NW
RB = 32
NBLK = CHUNK // RB
LANES = 16


def _sc_partials(x, run_lo, run_n):
    mesh = plsc.VectorSubcoreMesh(core_axis_name="c", subcore_axis_name="s")

    @functools.partial(
        pl.kernel,
        out_type=jax.ShapeDtypeStruct((NW * B, D), jnp.float32),
        mesh=mesh,
        scratch_types=[
            pltpu.VMEM((2, RB, D), jnp.float32),
            pltpu.VMEM((B, D), jnp.float32),
            pltpu.VMEM((LANES,), jnp.int32),
            pltpu.VMEM((LANES,), jnp.int32),
            pltpu.SMEM((B,), jnp.int32),
            pltpu.SMEM((B,), jnp.int32),
            pltpu.SemaphoreType.DMA,
        ],
        compiler_params=pltpu.CompilerParams(needs_layout_passes=False),
    )
    def k(x_hbm, lo_hbm, n_hbm, out_hbm, buf, acc, lo_v, n_v, lo_s, n_s, sem):
        c = lax.axis_index("c")
        s = lax.axis_index("s")
        w = s * NC + c
        base = w * CHUNK

        pltpu.sync_copy(lo_hbm.at[pl.ds(w * B, B)], lo_v)
        pltpu.sync_copy(n_hbm.at[pl.ds(w * B, B)], n_v)

        lanes = lax.iota(jnp.int32, LANES)
        lo_all = lo_v[...]
        n_all = n_v[...]

        def extract(j, _):
            lo_s[j] = jnp.sum(jnp.where(lanes == j, lo_all, 0)) - base
            n_s[j] = jnp.sum(jnp.where(lanes == j, n_all, 0))
            return 0

        lax.fori_loop(0, B, extract, 0)

        zeros = jnp.zeros((LANES,), jnp.float32)

        def zero_b(b, _):
            def zero_cs(cs, __):
                acc[b, pl.ds(cs * LANES, LANES)] = zeros
                return 0

            return lax.fori_loop(0, D // LANES, zero_cs, 0)

        lax.fori_loop(0, B, zero_b, 0)

        def block_copy(blk, slot):
            return pltpu.make_async_copy(
                x_hbm.at[pl.ds(base + blk * RB, RB), :],
                buf.at[slot],
                sem,
            )

        block_copy(0, 0).start()
        UNROLL = 8

        def blk_body(blk, _):
            slot = lax.rem(blk, 2)
            block_copy(blk, slot).wait()

            @pl.when(blk + 1 < NBLK)
            def _():
                block_copy(blk + 1, 1 - slot).start()

            wlo = blk * RB

            def j_body(j, __):
                lo_j = lo_s[j]
                n_j = n_s[j]
                lo_w = jnp.maximum(lo_j, wlo)
                hi_w = jnp.minimum(lo_j + n_j, wlo + RB)
                n_w = hi_w - lo_w
                r0 = lo_w - wlo

                @pl.when(n_w > 0)
                def _():
                    nu = n_w - lax.rem(n_w, UNROLL)

                    def cs_body(cs, __):
                        c0 = cs * (2 * LANES)

                        def body_u(kk, carry):
                            a0, a1 = carry
                            r = r0 + kk * UNROLL
                            for t in range(UNROLL):
                                a0 = a0 + buf[slot, r + t, pl.ds(c0, LANES)]
                                a1 = a1 + buf[
                                    slot, r + t, pl.ds(c0 + LANES, LANES)
                                ]
                            return (a0, a1)

                        a0, a1 = lax.fori_loop(
                            0, nu // UNROLL, body_u, (zeros, zeros)
                        )

                        def body_rem(i, carry):
                            a0, a1 = carry
                            a0 = a0 + buf[slot, r0 + i, pl.ds(c0, LANES)]
                            a1 = a1 + buf[
                                slot, r0 + i, pl.ds(c0 + LANES, LANES)
                            ]
                            return (a0, a1)

                        a0, a1 = lax.fori_loop(nu, n_w, body_rem, (a0, a1))
                        plsc.addupdate(acc.at[j, pl.ds(c0, LANES)], a0)
                        plsc.addupdate(acc.at[j, pl.ds(c0 + LANES, LANES)], a1)
                        return 0

                    lax.fori_loop(0, D // (2 * LANES), cs_body, 0)

                return 0

            return lax.fori_loop(0, B, j_body, 0)

        lax.fori_loop(0, NBLK, blk_body, 0)

        pltpu.sync_copy(acc, out_hbm.at[pl.ds(w * B, B), :])

    return k(x, run_lo, run_n)


def _combine(partials, inv_len):
    def body(p_ref, inv_ref, o_ref):
        o_ref[...] = (
            jnp.sum(p_ref[...].reshape(NW, B, D), axis=0) * inv_ref[...]
        )

    return pl.pallas_call(
        body,
        out_shape=jax.ShapeDtypeStruct((B, D), jnp.float32),
    )(partials, inv_len)


def kernel(x, batch_lengths):
    ends = jnp.cumsum(batch_lengths, dtype=jnp.int32)
    starts = jnp.concatenate([jnp.zeros((1,), jnp.int32), ends[:-1]])

    wlo = jnp.arange(NW, dtype=jnp.int32)[:, None] * CHUNK
    whi = wlo + CHUNK
    lo = jnp.maximum(starts[None, :], wlo)
    hi = jnp.minimum(ends[None, :], whi)
    n = jnp.maximum(hi - lo, 0)

    partials = _sc_partials(x, lo.reshape(-1), n.reshape(-1))
    inv_len = (1.0 / batch_lengths.astype(jnp.float32))[:, None]
    return _combine(partials, inv_len)
